# baseline (speedup 1.0000x reference)
import functools

import jax
import jax.numpy as jnp
from jax import lax
from jax.experimental import pallas as pl
from jax.experimental.pallas import tpu as pltpu
from jax.experimental.pallas import tpu_sc as plsc

_ROWS, _COLS = 16384, 128

_SC_ROWS = 0
_TC_ROWS = _ROWS - _SC_ROWS


_BLOCK_ROWS = 2048
_TILE = 128
_NCHAIN = 1
_STRIDE = _TILE * _NCHAIN


_LOG2E = 1.4426950408889634
_LN2 = 0.6931471805599453


def _bce_block(x, t):
    return jnp.maximum(x, 0.0) - x * t + jnp.log1p(jnp.exp(-jnp.abs(x)))


def _tc_kernel(preds_ref, targets_ref, out_ref, acc_ref):
    i = pl.program_id(0)

    def body(j, accs):
        base = j * _STRIDE
        new = []
        for c in range(_NCHAIN):
            a, b = accs[c]
            x = preds_ref[pl.ds(base + c * _TILE, _TILE), :]
            t = targets_ref[pl.ds(base + c * _TILE, _TILE), :]
            e = jnp.exp2(jnp.abs(x) * -_LOG2E)
            a = a + (jnp.maximum(x, 0.0) - x * t)
            b = b + jnp.log2(1.0 + e)
            new.append((a, b))
        return tuple(new)

    z = jnp.zeros((_TILE, _COLS), jnp.float32)
    zeros = tuple((z, z) for _ in range(_NCHAIN))
    accs = lax.fori_loop(0, _BLOCK_ROWS // _STRIDE, body, zeros)
    tot_a = sum((ab[0] for ab in accs[1:]), accs[0][0])
    tot_b = sum((ab[1] for ab in accs[1:]), accs[0][1])
    tot = tot_a + jnp.float32(_LN2) * tot_b

    @pl.when(i == 0)
    def _init():
        acc_ref[...] = tot

    @pl.when(i > 0)
    def _acc():
        acc_ref[...] += tot

    @pl.when(i == pl.num_programs(0) - 1)
    def _fin():
        out_ref[0] = jnp.sum(acc_ref[...])


def _tc_partial(preds, targets):
    grid = _TC_ROWS // _BLOCK_ROWS
    out = pl.pallas_call(
        _tc_kernel,
        grid=(grid,),
        in_specs=[
            pl.BlockSpec((_BLOCK_ROWS, _COLS), lambda i: (i, 0)),
            pl.BlockSpec((_BLOCK_ROWS, _COLS), lambda i: (i, 0)),
        ],
        out_specs=pl.BlockSpec(memory_space=pltpu.SMEM),
        out_shape=jax.ShapeDtypeStruct((1,), jnp.float32),
        scratch_shapes=[pltpu.VMEM((_TILE, _COLS), jnp.float32)],
    )(preds, targets)
    return out[0]


_NW = 32
_SC_CHUNK = 256

_LOG1P_C = (1.92163718e-07, 9.99973120e-01, -4.99379939e-01, 3.27787496e-01,
            -2.24754522e-01, 1.33144468e-01, -5.41071155e-02, 1.04836725e-02)


def _log1p_poly(e):
    acc = jnp.full(e.shape, _LOG1P_C[7], jnp.float32)
    for k in range(6, -1, -1):
        acc = acc * e + jnp.float32(_LOG1P_C[k])
    return acc


def _sc_bce_block(x, t):
    e = jnp.exp(-jnp.abs(x))
    return jnp.maximum(x, 0.0) - x * t + _log1p_poly(e)


def _sc_partial(preds, targets):
    rows_w = _SC_ROWS // _NW
    nchunk = rows_w // _SC_CHUNK
    mesh = plsc.VectorSubcoreMesh(core_axis_name="c", subcore_axis_name="s")

    @functools.partial(
        pl.kernel, mesh=mesh,
        out_type=jax.ShapeDtypeStruct((_NW, 16), jnp.float32),
        scratch_types=[
            pltpu.VMEM((_SC_CHUNK, _COLS), jnp.float32),
            pltpu.VMEM((_SC_CHUNK, _COLS), jnp.float32),
            pltpu.VMEM((16,), jnp.float32),
        ],
    )
    def k(preds_hbm, targets_hbm, out_hbm, xbuf, tbuf, accbuf):
        wid = lax.axis_index("s") * 2 + lax.axis_index("c")

        def chunk_body(ci, acc):
            base = _TC_ROWS + wid * rows_w + ci * _SC_CHUNK
            pltpu.sync_copy(preds_hbm.at[pl.ds(base, _SC_CHUNK)], xbuf)
            pltpu.sync_copy(targets_hbm.at[pl.ds(base, _SC_CHUNK)], tbuf)

            def row_body(i, acc):
                for j in range(_COLS // 16):
                    x = xbuf[i, pl.ds(j * 16, 16)]
                    t = tbuf[i, pl.ds(j * 16, 16)]
                    acc = acc + _sc_bce_block(x, t)
                return acc

            return lax.fori_loop(0, _SC_CHUNK, row_body, acc)

        acc = lax.fori_loop(0, nchunk, chunk_body,
                            jnp.zeros((16,), jnp.float32))
        accbuf[...] = acc
        pltpu.sync_copy(accbuf, out_hbm.at[wid])

    return jnp.sum(k(preds, targets))


def kernel(preds, targets):
    parts = []
    if _TC_ROWS:
        parts.append(_tc_partial(preds, targets))
    if _SC_ROWS:
        parts.append(_sc_partial(preds, targets))
    return sum(parts) * (1.0 / (_ROWS * _COLS))


# speedup vs baseline: 1.2605x; 1.2605x over previous
import functools

import jax
import jax.numpy as jnp
from jax import lax
from jax.experimental import pallas as pl
from jax.experimental.pallas import tpu as pltpu
from jax.experimental.pallas import tpu_sc as plsc

_ROWS, _COLS = 16384, 128

_SC_ROWS = 0
_TC_ROWS = _ROWS - _SC_ROWS


_BLOCK_ROWS = 4096
_TILE = 128
_NCHAIN = 1
_STRIDE = _TILE * _NCHAIN


_LOG2E = 1.4426950408889634
_LN2 = 0.6931471805599453


def _bce_block(x, t):
    return jnp.maximum(x, 0.0) - x * t + jnp.log1p(jnp.exp(-jnp.abs(x)))


def _tc_kernel(preds_ref, targets_ref, out_ref, acc_ref):
    i = pl.program_id(0)

    def body(j, accs):
        base = j * _STRIDE
        new = []
        for c in range(_NCHAIN):
            a, b = accs[c]
            x = preds_ref[pl.ds(base + c * _TILE, _TILE), :]
            t = targets_ref[pl.ds(base + c * _TILE, _TILE), :]
            e = jnp.exp2(jnp.abs(x) * -_LOG2E)
            a = a + (jnp.maximum(x, 0.0) - x * t)
            b = b + jnp.log2(1.0 + e)
            new.append((a, b))
        return tuple(new)

    z = jnp.zeros((_TILE, _COLS), jnp.float32)
    zeros = tuple((z, z) for _ in range(_NCHAIN))
    accs = lax.fori_loop(0, _BLOCK_ROWS // _STRIDE, body, zeros)
    tot_a = sum((ab[0] for ab in accs[1:]), accs[0][0])
    tot_b = sum((ab[1] for ab in accs[1:]), accs[0][1])
    tot = tot_a + jnp.float32(_LN2) * tot_b

    @pl.when(i == 0)
    def _init():
        acc_ref[...] = tot

    @pl.when(i > 0)
    def _acc():
        acc_ref[...] += tot

    @pl.when(i == pl.num_programs(0) - 1)
    def _fin():
        out_ref[0] = jnp.sum(acc_ref[...])


def _tc_partial(preds, targets):
    grid = _TC_ROWS // _BLOCK_ROWS
    out = pl.pallas_call(
        _tc_kernel,
        grid=(grid,),
        in_specs=[
            pl.BlockSpec((_BLOCK_ROWS, _COLS), lambda i: (i, 0)),
            pl.BlockSpec((_BLOCK_ROWS, _COLS), lambda i: (i, 0)),
        ],
        out_specs=pl.BlockSpec(memory_space=pltpu.SMEM),
        out_shape=jax.ShapeDtypeStruct((1,), jnp.float32),
        scratch_shapes=[pltpu.VMEM((_TILE, _COLS), jnp.float32)],
    )(preds, targets)
    return out[0]


_CH = 2048
_NCHK = _ROWS // _CH
_NBUF = 3


def _tc_kernel_manual(p_hbm, t_hbm, out_ref, pbuf, tbuf, psem, tsem):
    def dma_p(ci, slot):
        return pltpu.make_async_copy(
            p_hbm.at[pl.ds(ci * _CH, _CH), :], pbuf.at[slot], psem.at[slot])

    def dma_t(ci, slot):
        return pltpu.make_async_copy(
            t_hbm.at[pl.ds(ci * _CH, _CH), :], tbuf.at[slot], tsem.at[slot])

    for s in range(min(_NBUF, _NCHK)):
        dma_p(s, s).start()
        dma_t(s, s).start()

    z = jnp.zeros((_TILE, _COLS), jnp.float32)
    accs = (z, z)
    for ci in range(_NCHK):
        slot = ci % _NBUF
        dma_p(ci, slot).wait()
        dma_t(ci, slot).wait()

        def sub(j, accs, slot=slot):
            a, b = accs
            x = pbuf[slot, pl.ds(j * _TILE, _TILE), :]
            t = tbuf[slot, pl.ds(j * _TILE, _TILE), :]
            e = jnp.exp2(jnp.abs(x) * -_LOG2E)
            a = a + (jnp.maximum(x, 0.0) - x * t)
            b = b + jnp.log2(1.0 + e)
            return (a, b)

        accs = lax.fori_loop(0, _CH // _TILE, sub, accs)
        if ci + _NBUF < _NCHK:
            dma_p(ci + _NBUF, slot).start()
            dma_t(ci + _NBUF, slot).start()

    out_ref[0] = jnp.sum(accs[0]) + jnp.float32(_LN2) * jnp.sum(accs[1])


def _tc_partial_manual(preds, targets):
    out = pl.pallas_call(
        _tc_kernel_manual,
        in_specs=[
            pl.BlockSpec(memory_space=pl.ANY),
            pl.BlockSpec(memory_space=pl.ANY),
        ],
        out_specs=pl.BlockSpec(memory_space=pltpu.SMEM),
        out_shape=jax.ShapeDtypeStruct((1,), jnp.float32),
        scratch_shapes=[
            pltpu.VMEM((_NBUF, _CH, _COLS), jnp.float32),
            pltpu.VMEM((_NBUF, _CH, _COLS), jnp.float32),
            pltpu.SemaphoreType.DMA((_NBUF,)),
            pltpu.SemaphoreType.DMA((_NBUF,)),
        ],
    )(preds, targets)
    return out[0]


_NW = 32
_SC_CHUNK = 256

_LOG1P_C = (1.92163718e-07, 9.99973120e-01, -4.99379939e-01, 3.27787496e-01,
            -2.24754522e-01, 1.33144468e-01, -5.41071155e-02, 1.04836725e-02)


def _log1p_poly(e):
    acc = jnp.full(e.shape, _LOG1P_C[7], jnp.float32)
    for k in range(6, -1, -1):
        acc = acc * e + jnp.float32(_LOG1P_C[k])
    return acc


def _sc_bce_block(x, t):
    e = jnp.exp(-jnp.abs(x))
    return jnp.maximum(x, 0.0) - x * t + _log1p_poly(e)


def _sc_partial(preds, targets):
    rows_w = _SC_ROWS // _NW
    nchunk = rows_w // _SC_CHUNK
    mesh = plsc.VectorSubcoreMesh(core_axis_name="c", subcore_axis_name="s")

    @functools.partial(
        pl.kernel, mesh=mesh,
        out_type=jax.ShapeDtypeStruct((_NW, 16), jnp.float32),
        scratch_types=[
            pltpu.VMEM((_SC_CHUNK, _COLS), jnp.float32),
            pltpu.VMEM((_SC_CHUNK, _COLS), jnp.float32),
            pltpu.VMEM((16,), jnp.float32),
        ],
    )
    def k(preds_hbm, targets_hbm, out_hbm, xbuf, tbuf, accbuf):
        wid = lax.axis_index("s") * 2 + lax.axis_index("c")

        def chunk_body(ci, acc):
            base = _TC_ROWS + wid * rows_w + ci * _SC_CHUNK
            pltpu.sync_copy(preds_hbm.at[pl.ds(base, _SC_CHUNK)], xbuf)
            pltpu.sync_copy(targets_hbm.at[pl.ds(base, _SC_CHUNK)], tbuf)

            def row_body(i, acc):
                for j in range(_COLS // 16):
                    x = xbuf[i, pl.ds(j * 16, 16)]
                    t = tbuf[i, pl.ds(j * 16, 16)]
                    acc = acc + _sc_bce_block(x, t)
                return acc

            return lax.fori_loop(0, _SC_CHUNK, row_body, acc)

        acc = lax.fori_loop(0, nchunk, chunk_body,
                            jnp.zeros((16,), jnp.float32))
        accbuf[...] = acc
        pltpu.sync_copy(accbuf, out_hbm.at[wid])

    return jnp.sum(k(preds, targets))


def kernel(preds, targets):
    parts = []
    if _TC_ROWS:
        parts.append(_tc_partial_manual(preds, targets))
    if _SC_ROWS:
        parts.append(_sc_partial(preds, targets))
    return sum(parts) * (1.0 / (_ROWS * _COLS))


# speedup vs baseline: 1.3240x; 1.0504x over previous
import functools

import jax
import jax.numpy as jnp
from jax import lax
from jax.experimental import pallas as pl
from jax.experimental.pallas import tpu as pltpu
from jax.experimental.pallas import tpu_sc as plsc

_ROWS, _COLS = 16384, 128

_SC_ROWS = 0
_TC_ROWS = _ROWS - _SC_ROWS


_BLOCK_ROWS = 4096
_TILE = 128
_NCHAIN = 1
_STRIDE = _TILE * _NCHAIN


_LOG2E = 1.4426950408889634
_LN2 = 0.6931471805599453


def _bce_block(x, t):
    return jnp.maximum(x, 0.0) - x * t + jnp.log1p(jnp.exp(-jnp.abs(x)))


def _tc_kernel(preds_ref, targets_ref, out_ref, acc_ref):
    i = pl.program_id(0)

    def body(j, accs):
        base = j * _STRIDE
        new = []
        for c in range(_NCHAIN):
            a, b = accs[c]
            x = preds_ref[pl.ds(base + c * _TILE, _TILE), :]
            t = targets_ref[pl.ds(base + c * _TILE, _TILE), :]
            e = jnp.exp2(jnp.abs(x) * -_LOG2E)
            a = a + (jnp.maximum(x, 0.0) - x * t)
            b = b + jnp.log2(1.0 + e)
            new.append((a, b))
        return tuple(new)

    z = jnp.zeros((_TILE, _COLS), jnp.float32)
    zeros = tuple((z, z) for _ in range(_NCHAIN))
    accs = lax.fori_loop(0, _BLOCK_ROWS // _STRIDE, body, zeros)
    tot_a = sum((ab[0] for ab in accs[1:]), accs[0][0])
    tot_b = sum((ab[1] for ab in accs[1:]), accs[0][1])
    tot = tot_a + jnp.float32(_LN2) * tot_b

    @pl.when(i == 0)
    def _init():
        acc_ref[...] = tot

    @pl.when(i > 0)
    def _acc():
        acc_ref[...] += tot

    @pl.when(i == pl.num_programs(0) - 1)
    def _fin():
        out_ref[0] = jnp.sum(acc_ref[...])


def _tc_partial(preds, targets):
    grid = _TC_ROWS // _BLOCK_ROWS
    out = pl.pallas_call(
        _tc_kernel,
        grid=(grid,),
        in_specs=[
            pl.BlockSpec((_BLOCK_ROWS, _COLS), lambda i: (i, 0)),
            pl.BlockSpec((_BLOCK_ROWS, _COLS), lambda i: (i, 0)),
        ],
        out_specs=pl.BlockSpec(memory_space=pltpu.SMEM),
        out_shape=jax.ShapeDtypeStruct((1,), jnp.float32),
        scratch_shapes=[pltpu.VMEM((_TILE, _COLS), jnp.float32)],
    )(preds, targets)
    return out[0]


_CH = 2048
_NCHK = _ROWS // _CH
_NBUF = 8


def _tc_kernel_manual(p_hbm, t_hbm, out_ref, pbuf, tbuf, psem, tsem):
    def dma_p(ci, slot):
        return pltpu.make_async_copy(
            p_hbm.at[pl.ds(ci * _CH, _CH), :], pbuf.at[slot], psem.at[slot])

    def dma_t(ci, slot):
        return pltpu.make_async_copy(
            t_hbm.at[pl.ds(ci * _CH, _CH), :], tbuf.at[slot], tsem.at[slot])

    for s in range(min(_NBUF, _NCHK)):
        dma_p(s, s).start()
        dma_t(s, s).start()

    z = jnp.zeros((_TILE, _COLS), jnp.float32)
    accs = (z, z)
    for ci in range(_NCHK):
        slot = ci % _NBUF
        dma_p(ci, slot).wait()
        dma_t(ci, slot).wait()

        def sub(j, accs, slot=slot):
            a, b = accs
            x = pbuf[slot, pl.ds(j * _TILE, _TILE), :]
            t = tbuf[slot, pl.ds(j * _TILE, _TILE), :]
            e = jnp.exp2(jnp.abs(x) * -_LOG2E)
            a = a + (jnp.maximum(x, 0.0) - x * t)
            b = b + jnp.log2(1.0 + e)
            return (a, b)

        accs = lax.fori_loop(0, _CH // _TILE, sub, accs)
        if ci + _NBUF < _NCHK:
            dma_p(ci + _NBUF, slot).start()
            dma_t(ci + _NBUF, slot).start()

    out_ref[0] = jnp.sum(accs[0]) + jnp.float32(_LN2) * jnp.sum(accs[1])


def _tc_partial_manual(preds, targets):
    out = pl.pallas_call(
        _tc_kernel_manual,
        in_specs=[
            pl.BlockSpec(memory_space=pl.ANY),
            pl.BlockSpec(memory_space=pl.ANY),
        ],
        out_specs=pl.BlockSpec(memory_space=pltpu.SMEM),
        out_shape=jax.ShapeDtypeStruct((1,), jnp.float32),
        scratch_shapes=[
            pltpu.VMEM((_NBUF, _CH, _COLS), jnp.float32),
            pltpu.VMEM((_NBUF, _CH, _COLS), jnp.float32),
            pltpu.SemaphoreType.DMA((_NBUF,)),
            pltpu.SemaphoreType.DMA((_NBUF,)),
        ],
    )(preds, targets)
    return out[0]


_NW = 32
_SC_CHUNK = 256

_LOG1P_C = (1.92163718e-07, 9.99973120e-01, -4.99379939e-01, 3.27787496e-01,
            -2.24754522e-01, 1.33144468e-01, -5.41071155e-02, 1.04836725e-02)


def _log1p_poly(e):
    acc = jnp.full(e.shape, _LOG1P_C[7], jnp.float32)
    for k in range(6, -1, -1):
        acc = acc * e + jnp.float32(_LOG1P_C[k])
    return acc


def _sc_bce_block(x, t):
    e = jnp.exp(-jnp.abs(x))
    return jnp.maximum(x, 0.0) - x * t + _log1p_poly(e)


def _sc_partial(preds, targets):
    rows_w = _SC_ROWS // _NW
    nchunk = rows_w // _SC_CHUNK
    mesh = plsc.VectorSubcoreMesh(core_axis_name="c", subcore_axis_name="s")

    @functools.partial(
        pl.kernel, mesh=mesh,
        out_type=jax.ShapeDtypeStruct((_NW, 16), jnp.float32),
        scratch_types=[
            pltpu.VMEM((_SC_CHUNK, _COLS), jnp.float32),
            pltpu.VMEM((_SC_CHUNK, _COLS), jnp.float32),
            pltpu.VMEM((16,), jnp.float32),
        ],
    )
    def k(preds_hbm, targets_hbm, out_hbm, xbuf, tbuf, accbuf):
        wid = lax.axis_index("s") * 2 + lax.axis_index("c")

        def chunk_body(ci, acc):
            base = _TC_ROWS + wid * rows_w + ci * _SC_CHUNK
            pltpu.sync_copy(preds_hbm.at[pl.ds(base, _SC_CHUNK)], xbuf)
            pltpu.sync_copy(targets_hbm.at[pl.ds(base, _SC_CHUNK)], tbuf)

            def row_body(i, acc):
                for j in range(_COLS // 16):
                    x = xbuf[i, pl.ds(j * 16, 16)]
                    t = tbuf[i, pl.ds(j * 16, 16)]
                    acc = acc + _sc_bce_block(x, t)
                return acc

            return lax.fori_loop(0, _SC_CHUNK, row_body, acc)

        acc = lax.fori_loop(0, nchunk, chunk_body,
                            jnp.zeros((16,), jnp.float32))
        accbuf[...] = acc
        pltpu.sync_copy(accbuf, out_hbm.at[wid])

    return jnp.sum(k(preds, targets))


def kernel(preds, targets):
    parts = []
    if _TC_ROWS:
        parts.append(_tc_partial_manual(preds, targets))
    if _SC_ROWS:
        parts.append(_sc_partial(preds, targets))
    return sum(parts) * (1.0 / (_ROWS * _COLS))


# speedup vs baseline: 1.3497x; 1.0194x over previous
import functools

import jax
import jax.numpy as jnp
from jax import lax
from jax.experimental import pallas as pl
from jax.experimental.pallas import tpu as pltpu
from jax.experimental.pallas import tpu_sc as plsc

_ROWS, _COLS = 16384, 128

_SC_ROWS = 0
_TC_ROWS = _ROWS - _SC_ROWS


_BLOCK_ROWS = 4096
_TILE = 128
_NCHAIN = 1
_STRIDE = _TILE * _NCHAIN


_LOG2E = 1.4426950408889634
_LN2 = 0.6931471805599453


def _bce_block(x, t):
    return jnp.maximum(x, 0.0) - x * t + jnp.log1p(jnp.exp(-jnp.abs(x)))


def _tc_kernel(preds_ref, targets_ref, out_ref, acc_ref):
    i = pl.program_id(0)

    def body(j, accs):
        base = j * _STRIDE
        new = []
        for c in range(_NCHAIN):
            a, b = accs[c]
            x = preds_ref[pl.ds(base + c * _TILE, _TILE), :]
            t = targets_ref[pl.ds(base + c * _TILE, _TILE), :]
            e = jnp.exp2(jnp.abs(x) * -_LOG2E)
            a = a + (jnp.maximum(x, 0.0) - x * t)
            b = b + jnp.log2(1.0 + e)
            new.append((a, b))
        return tuple(new)

    z = jnp.zeros((_TILE, _COLS), jnp.float32)
    zeros = tuple((z, z) for _ in range(_NCHAIN))
    accs = lax.fori_loop(0, _BLOCK_ROWS // _STRIDE, body, zeros)
    tot_a = sum((ab[0] for ab in accs[1:]), accs[0][0])
    tot_b = sum((ab[1] for ab in accs[1:]), accs[0][1])
    tot = tot_a + jnp.float32(_LN2) * tot_b

    @pl.when(i == 0)
    def _init():
        acc_ref[...] = tot

    @pl.when(i > 0)
    def _acc():
        acc_ref[...] += tot

    @pl.when(i == pl.num_programs(0) - 1)
    def _fin():
        out_ref[0] = jnp.sum(acc_ref[...])


def _tc_partial(preds, targets):
    grid = _TC_ROWS // _BLOCK_ROWS
    out = pl.pallas_call(
        _tc_kernel,
        grid=(grid,),
        in_specs=[
            pl.BlockSpec((_BLOCK_ROWS, _COLS), lambda i: (i, 0)),
            pl.BlockSpec((_BLOCK_ROWS, _COLS), lambda i: (i, 0)),
        ],
        out_specs=pl.BlockSpec(memory_space=pltpu.SMEM),
        out_shape=jax.ShapeDtypeStruct((1,), jnp.float32),
        scratch_shapes=[pltpu.VMEM((_TILE, _COLS), jnp.float32)],
    )(preds, targets)
    return out[0]


_CH = 1024
_NCHK = _ROWS // _CH
_NBUF = 16


def _tc_kernel_manual(p_hbm, t_hbm, out_ref, pbuf, tbuf, psem, tsem):
    def dma_p(ci, slot):
        return pltpu.make_async_copy(
            p_hbm.at[pl.ds(ci * _CH, _CH), :], pbuf.at[slot], psem.at[slot])

    def dma_t(ci, slot):
        return pltpu.make_async_copy(
            t_hbm.at[pl.ds(ci * _CH, _CH), :], tbuf.at[slot], tsem.at[slot])

    for s in range(min(_NBUF, _NCHK)):
        dma_p(s, s).start()
        dma_t(s, s).start()

    z = jnp.zeros((_TILE, _COLS), jnp.float32)
    accs = (z, z)
    for ci in range(_NCHK):
        slot = ci % _NBUF
        dma_p(ci, slot).wait()
        dma_t(ci, slot).wait()

        def sub(j, accs, slot=slot):
            a, b = accs
            x = pbuf[slot, pl.ds(j * _TILE, _TILE), :]
            t = tbuf[slot, pl.ds(j * _TILE, _TILE), :]
            e = jnp.exp2(jnp.abs(x) * -_LOG2E)
            a = a + (jnp.maximum(x, 0.0) - x * t)
            b = b + jnp.log2(1.0 + e)
            return (a, b)

        accs = lax.fori_loop(0, _CH // _TILE, sub, accs)
        if ci + _NBUF < _NCHK:
            dma_p(ci + _NBUF, slot).start()
            dma_t(ci + _NBUF, slot).start()

    out_ref[0] = jnp.sum(accs[0]) + jnp.float32(_LN2) * jnp.sum(accs[1])


def _tc_partial_manual(preds, targets):
    out = pl.pallas_call(
        _tc_kernel_manual,
        in_specs=[
            pl.BlockSpec(memory_space=pl.ANY),
            pl.BlockSpec(memory_space=pl.ANY),
        ],
        out_specs=pl.BlockSpec(memory_space=pltpu.SMEM),
        out_shape=jax.ShapeDtypeStruct((1,), jnp.float32),
        scratch_shapes=[
            pltpu.VMEM((_NBUF, _CH, _COLS), jnp.float32),
            pltpu.VMEM((_NBUF, _CH, _COLS), jnp.float32),
            pltpu.SemaphoreType.DMA((_NBUF,)),
            pltpu.SemaphoreType.DMA((_NBUF,)),
        ],
    )(preds, targets)
    return out[0]


_NW = 32
_SC_CHUNK = 256

_LOG1P_C = (1.92163718e-07, 9.99973120e-01, -4.99379939e-01, 3.27787496e-01,
            -2.24754522e-01, 1.33144468e-01, -5.41071155e-02, 1.04836725e-02)


def _log1p_poly(e):
    acc = jnp.full(e.shape, _LOG1P_C[7], jnp.float32)
    for k in range(6, -1, -1):
        acc = acc * e + jnp.float32(_LOG1P_C[k])
    return acc


def _sc_bce_block(x, t):
    e = jnp.exp(-jnp.abs(x))
    return jnp.maximum(x, 0.0) - x * t + _log1p_poly(e)


def _sc_partial(preds, targets):
    rows_w = _SC_ROWS // _NW
    nchunk = rows_w // _SC_CHUNK
    mesh = plsc.VectorSubcoreMesh(core_axis_name="c", subcore_axis_name="s")

    @functools.partial(
        pl.kernel, mesh=mesh,
        out_type=jax.ShapeDtypeStruct((_NW, 16), jnp.float32),
        scratch_types=[
            pltpu.VMEM((_SC_CHUNK, _COLS), jnp.float32),
            pltpu.VMEM((_SC_CHUNK, _COLS), jnp.float32),
            pltpu.VMEM((16,), jnp.float32),
        ],
    )
    def k(preds_hbm, targets_hbm, out_hbm, xbuf, tbuf, accbuf):
        wid = lax.axis_index("s") * 2 + lax.axis_index("c")

        def chunk_body(ci, acc):
            base = _TC_ROWS + wid * rows_w + ci * _SC_CHUNK
            pltpu.sync_copy(preds_hbm.at[pl.ds(base, _SC_CHUNK)], xbuf)
            pltpu.sync_copy(targets_hbm.at[pl.ds(base, _SC_CHUNK)], tbuf)

            def row_body(i, acc):
                for j in range(_COLS // 16):
                    x = xbuf[i, pl.ds(j * 16, 16)]
                    t = tbuf[i, pl.ds(j * 16, 16)]
                    acc = acc + _sc_bce_block(x, t)
                return acc

            return lax.fori_loop(0, _SC_CHUNK, row_body, acc)

        acc = lax.fori_loop(0, nchunk, chunk_body,
                            jnp.zeros((16,), jnp.float32))
        accbuf[...] = acc
        pltpu.sync_copy(accbuf, out_hbm.at[wid])

    return jnp.sum(k(preds, targets))


def kernel(preds, targets):
    parts = []
    if _TC_ROWS:
        parts.append(_tc_partial_manual(preds, targets))
    if _SC_ROWS:
        parts.append(_sc_partial(preds, targets))
    return sum(parts) * (1.0 / (_ROWS * _COLS))
